# baseline (speedup 1.0000x reference)
import jax
import jax.numpy as jnp
from jax.experimental import pallas as pl

_G = 512


def _gcn_block(x_ref, adj_ref, w_ref, b_ref, out_ref):
    g, n, d = x_ref.shape
    o = w_ref.shape[1]
    x = x_ref[...].reshape(g * n, d)
    xw = jnp.dot(x, w_ref[...], preferred_element_type=jnp.float32)
    adj_f = adj_ref[...].astype(jnp.float32)
    dinv = jax.lax.rsqrt(jnp.sum(adj_f, axis=-1) + 1.0)
    xwn = xw.reshape(g, n, o) * dinv[:, :, None]
    agg = jax.lax.dot_general(
        adj_f, xwn, (((2,), (1,)), ((0,), (0,))),
        preferred_element_type=jnp.float32) + xwn
    out = agg * dinv[:, :, None] + b_ref[0][None, None, :]
    out_ref[...] = out.reshape(g, n * o)


def kernel(node_states, adj, W_gnn, b_gnn):
    b, n, d = node_states.shape
    o = W_gnn.shape[1]
    out = pl.pallas_call(
        _gcn_block,
        grid=(b // _G,),
        in_specs=[
            pl.BlockSpec((_G, n, d), lambda i: (i, 0, 0)),
            pl.BlockSpec((_G, n, n), lambda i: (i, 0, 0)),
            pl.BlockSpec((d, o), lambda i: (0, 0)),
            pl.BlockSpec((1, o), lambda i: (0, 0)),
        ],
        out_specs=pl.BlockSpec((_G, n * o), lambda i: (i, 0)),
        out_shape=jax.ShapeDtypeStruct((b, n * o), jnp.float32),
    )(node_states, adj, W_gnn, b_gnn.reshape(1, o))
    return out


# speedup vs baseline: 1.0128x; 1.0128x over previous
import jax
import jax.numpy as jnp
from jax.experimental import pallas as pl

_G = 256
_S = 4


def _gcn_block(x_ref, a0_ref, a1_ref, a2_ref, a3_ref, w_ref, b_ref, out_ref):
    g, n, d = x_ref.shape
    o = w_ref.shape[1]
    x = x_ref[...].reshape(g * n, d)
    xw = jnp.dot(x, w_ref[...], preferred_element_type=jnp.float32)
    adj_f = jnp.concatenate(
        [a0_ref[...], a1_ref[...], a2_ref[...], a3_ref[...]], axis=0
    ).astype(jnp.float32)
    dinv = jax.lax.rsqrt(jnp.sum(adj_f, axis=-1) + 1.0)
    xwn = xw.reshape(g, n, o) * dinv[:, :, None]
    agg = jax.lax.dot_general(
        adj_f, xwn, (((2,), (1,)), ((0,), (0,))),
        preferred_element_type=jnp.float32) + xwn
    out = agg * dinv[:, :, None] + b_ref[0][None, None, :]
    out_ref[...] = out.reshape(g, n * o)


def kernel(node_states, adj, W_gnn, b_gnn):
    b, n, d = node_states.shape
    o = W_gnn.shape[1]
    gs = _G // _S
    adj_specs = [
        pl.BlockSpec((gs, n, n), lambda i, q=q: (i * _S + q, 0, 0))
        for q in range(_S)
    ]
    out = pl.pallas_call(
        _gcn_block,
        grid=(b // _G,),
        in_specs=[pl.BlockSpec((_G, n, d), lambda i: (i, 0, 0))]
        + adj_specs
        + [
            pl.BlockSpec((d, o), lambda i: (0, 0)),
            pl.BlockSpec((1, o), lambda i: (0, 0)),
        ],
        out_specs=pl.BlockSpec((_G, n * o), lambda i: (i, 0)),
        out_shape=jax.ShapeDtypeStruct((b, n * o), jnp.float32),
    )(node_states, adj, adj, adj, adj, W_gnn, b_gnn.reshape(1, o))
    return out


# speedup vs baseline: 1.3175x; 1.3009x over previous
import jax
import jax.numpy as jnp
from jax.experimental import pallas as pl

_G = 256


def _gcn_block(x_ref, adj_ref, w_ref, b_ref, out_ref):
    g, n, d = x_ref.shape
    o = w_ref.shape[1]
    x = x_ref[...].reshape(g * n, d)
    xw = jnp.dot(x, w_ref[...], preferred_element_type=jnp.float32)
    adj_f = adj_ref[...].astype(jnp.float32).reshape(g, n, n)
    dinv = jax.lax.rsqrt(jnp.sum(adj_f, axis=-1) + 1.0)
    xwn = xw.reshape(g, n, o) * dinv[:, :, None]
    agg = jax.lax.dot_general(
        adj_f, xwn, (((2,), (1,)), ((0,), (0,))),
        preferred_element_type=jnp.float32) + xwn
    out = agg * dinv[:, :, None] + b_ref[0][None, None, :]
    out_ref[...] = out.reshape(g, n * o)


def kernel(node_states, adj, W_gnn, b_gnn):
    b, n, d = node_states.shape
    o = W_gnn.shape[1]
    out = pl.pallas_call(
        _gcn_block,
        grid=(b // _G,),
        in_specs=[
            pl.BlockSpec((_G, n, d), lambda i: (i, 0, 0)),
            pl.BlockSpec((_G, n * n), lambda i: (i, 0)),
            pl.BlockSpec((d, o), lambda i: (0, 0)),
            pl.BlockSpec((1, o), lambda i: (0, 0)),
        ],
        out_specs=pl.BlockSpec((_G, n * o), lambda i: (i, 0)),
        out_shape=jax.ShapeDtypeStruct((b, n * o), jnp.float32),
    )(node_states, adj.reshape(b, n * n), W_gnn, b_gnn.reshape(1, o))
    return out


# speedup vs baseline: 1.4548x; 1.1042x over previous
import jax
import jax.numpy as jnp
from jax.experimental import pallas as pl

_G = 256


def _gcn_block(x_ref, adj_ref, w_ref, b_ref, out_ref):
    g, n, d = x_ref.shape
    o = w_ref.shape[1]
    x = x_ref[...].reshape(g * n, d)
    xw = jnp.dot(x, w_ref[...], preferred_element_type=jnp.float32)
    adj_f = adj_ref[...].astype(jnp.float32).reshape(g, n, n)
    deg = jax.lax.dot_general(
        adj_f, jnp.ones((g, n, o), jnp.float32), (((2,), (1,)), ((0,), (0,))),
        preferred_element_type=jnp.float32)
    dinv = jax.lax.rsqrt(deg + 1.0)
    xwn = xw.reshape(g, n, o) * dinv
    agg = jax.lax.dot_general(
        adj_f, xwn, (((2,), (1,)), ((0,), (0,))),
        preferred_element_type=jnp.float32) + xwn
    out = agg * dinv + b_ref[0][None, None, :]
    out_ref[...] = out.reshape(g, n * o)


def kernel(node_states, adj, W_gnn, b_gnn):
    b, n, d = node_states.shape
    o = W_gnn.shape[1]
    out = pl.pallas_call(
        _gcn_block,
        grid=(b // _G,),
        in_specs=[
            pl.BlockSpec((_G, n, d), lambda i: (i, 0, 0)),
            pl.BlockSpec((_G, n * n), lambda i: (i, 0)),
            pl.BlockSpec((d, o), lambda i: (0, 0)),
            pl.BlockSpec((1, o), lambda i: (0, 0)),
        ],
        out_specs=pl.BlockSpec((_G, n * o), lambda i: (i, 0)),
        out_shape=jax.ShapeDtypeStruct((b, n * o), jnp.float32),
    )(node_states, adj.reshape(b, n * n), W_gnn, b_gnn.reshape(1, o))
    return out
